# baseline (speedup 1.0000x reference)
import functools

import jax
import jax.numpy as jnp
from jax import lax
from jax.experimental import pallas as pl
from jax.experimental.pallas import tpu as pltpu
from jax.experimental.pallas import tpu_sc as plsc

N_NODES = 10000
D = 256
HALF = 128
ROWW = 144
NC = 2
NS = 16
B = 64
N_ACC = 10240
CHUNK = N_ACC // NS


def _sc_segment_sums(xcat, garr, tarr, zer, n_batches, mode="full"):
    mesh = plsc.VectorSubcoreMesh(core_axis_name="c", subcore_axis_name="s")

    @functools.partial(
        pl.kernel,
        out_type=jax.ShapeDtypeStruct((2, NC, N_ACC, ROWW), jnp.float32),
        mesh=mesh,
        scratch_types=[
            pltpu.VMEM_SHARED((N_ACC, ROWW), jnp.float32),
            pltpu.VMEM((n_batches, B), jnp.int32),
            pltpu.VMEM((n_batches, B), jnp.int32),
            pltpu.VMEM((B, ROWW), jnp.float32),
            pltpu.VMEM((B, ROWW), jnp.float32),
            pltpu.SemaphoreType.DMA,
            pltpu.SemaphoreType.DMA,
            pltpu.SemaphoreType.DMA,
            pltpu.SemaphoreType.DMA,
        ],
        compiler_params=pltpu.CompilerParams(use_tc_tiling_on_sc=False),
    )
    def k(xcat_hbm, garr_hbm, tarr_hbm, zer_hbm, out_hbm,
          acc_sh, gidx_v, tgt_v, rows0, rows1, sg0, sg1, ss0, ss1):
        c = lax.axis_index("c")
        s = lax.axis_index("s")
        row0 = pl.multiple_of(s * CHUNK, 8)
        pltpu.sync_copy(zer_hbm.at[pl.ds(0, CHUNK)],
                        acc_sh.at[pl.ds(row0, CHUNK)])
        plsc.subcore_barrier()
        for d in range(2):
            pltpu.sync_copy(garr_hbm.at[d, c, s], gidx_v)
            pltpu.sync_copy(tarr_hbm.at[d, s], tgt_v)

            if mode == "scatter":
                pltpu.sync_copy(zer_hbm.at[pl.ds(0, B)], rows0)
                pltpu.sync_copy(zer_hbm.at[pl.ds(0, B)], rows1)

            if mode != "scatter":
                pltpu.async_copy(xcat_hbm.at[gidx_v.at[0]], rows0, sg0)
                pltpu.async_copy(xcat_hbm.at[gidx_v.at[1]], rows1, sg1)

            def pair(i, carry):
                b = 2 * i
                if mode != "scatter":
                    pltpu.make_async_copy(xcat_hbm.at[gidx_v.at[b]], rows0, sg0).wait()
                if mode != "gather":
                    cs0 = pltpu.async_copy(rows0, acc_sh.at[tgt_v.at[b]], ss0, add=True)
                if mode != "scatter":
                    pltpu.make_async_copy(xcat_hbm.at[gidx_v.at[b + 1]], rows1, sg1).wait()
                if mode != "gather":
                    cs1 = pltpu.async_copy(rows1, acc_sh.at[tgt_v.at[b + 1]], ss1, add=True)
                    cs0.wait()

                if mode != "scatter":
                    @pl.when(b + 2 < n_batches)
                    def _():
                        pltpu.async_copy(xcat_hbm.at[gidx_v.at[b + 2]], rows0, sg0)
                if mode != "gather":
                    cs1.wait()

                if mode != "scatter":
                    @pl.when(b + 3 < n_batches)
                    def _():
                        pltpu.async_copy(xcat_hbm.at[gidx_v.at[b + 3]], rows1, sg1)
                return carry

            lax.fori_loop(0, n_batches // 2, pair, 0)
            plsc.subcore_barrier()
            pltpu.sync_copy(acc_sh.at[pl.ds(row0, CHUNK)],
                            out_hbm.at[d, c, pl.ds(row0, CHUNK)])
            if d == 0:
                pltpu.sync_copy(zer_hbm.at[pl.ds(0, CHUNK)],
                                acc_sh.at[pl.ds(row0, CHUNK)])
                plsc.subcore_barrier()

    return k(xcat, garr, tarr, zer)


def _tc_combine(agg, x, W_in, b_in, W_out, b_out, W_root, b_root):
    blk = 400
    grid = (N_NODES // blk,)

    def body(agg_ref, x_ref, wi_ref, wo_ref, wr_ref, b_ref, o_ref):
        dot = functools.partial(
            lax.dot_general,
            dimension_numbers=(((1,), (0,)), ((), ())),
            precision=lax.Precision.HIGHEST,
            preferred_element_type=jnp.float32,
        )
        acc = dot(x_ref[...], wr_ref[...])
        for d, w_ref in ((0, wi_ref), (1, wo_ref)):
            cnt = agg_ref[d, 0, :, HALF:HALF + 1]
            scale = 0.5 / jnp.maximum(cnt, 1.0)
            for c in range(2):
                h = agg_ref[d, c, :, 0:HALF] * scale
                acc = acc + dot(h, w_ref[c * HALF:(c + 1) * HALF, :])
        o_ref[...] = acc + b_ref[...]

    bias = (0.5 * b_in + 0.5 * b_out + b_root).reshape(1, D)
    return pl.pallas_call(
        body,
        grid=grid,
        in_specs=[
            pl.BlockSpec((2, NC, blk, ROWW), lambda i: (0, 0, i, 0)),

            pl.BlockSpec((blk, D), lambda i: (i, 0)),
            pl.BlockSpec((D, D), lambda i: (0, 0)),
            pl.BlockSpec((D, D), lambda i: (0, 0)),
            pl.BlockSpec((D, D), lambda i: (0, 0)),
            pl.BlockSpec((1, D), lambda i: (0, 0)),
        ],
        out_specs=pl.BlockSpec((blk, D), lambda i: (i, 0)),
        out_shape=jax.ShapeDtypeStruct((N_NODES, D), jnp.float32),
    )(agg, x, W_in, W_out, W_root, bias)


def kernel(x, edge_index, W_in, b_in, W_out, b_out, W_root, b_root):
    n_edges = edge_index.shape[1]
    src = edge_index[0].astype(jnp.int32)
    dst = edge_index[1].astype(jnp.int32)

    ones = jnp.ones((N_NODES, 1), jnp.float32)
    zpad = jnp.zeros((N_NODES, ROWW - HALF - 1), jnp.float32)
    xcat = jnp.concatenate([
        jnp.concatenate([x[:, :HALF], ones, zpad], axis=1),
        jnp.concatenate([x[:, HALF:], ones, zpad], axis=1),
        jnp.zeros((8, ROWW), jnp.float32),
    ], axis=0)

    ept = ((n_edges // NS) + 2 * B - 1) // (2 * B) * (2 * B)
    n_batches = ept // B
    e_pad = NS * ept
    pad = e_pad - n_edges
    pad_g = 2 * N_NODES + (jnp.arange(pad, dtype=jnp.int32) % 8)
    pad_t = jnp.arange(pad, dtype=jnp.int32) % N_NODES

    def tiled(idx, pad_vals):
        return jnp.concatenate([idx, pad_vals]).reshape(NS, n_batches, B)

    garr = jnp.stack([
        jnp.stack([tiled(src, pad_g), tiled(src + N_NODES, pad_g)]),
        jnp.stack([tiled(dst, pad_g), tiled(dst + N_NODES, pad_g)]),
    ])
    tarr = jnp.stack([tiled(dst, pad_t), tiled(src, pad_t)])

    zer = jnp.zeros((CHUNK, ROWW), jnp.float32)
    agg = _sc_segment_sums(xcat, garr, tarr, zer, n_batches)
    agg_g = _sc_segment_sums(xcat, garr, tarr, zer, n_batches, mode="gather")
    agg_s = _sc_segment_sums(xcat, garr, tarr, zer, n_batches, mode="scatter")
    agg = agg + 0.0 * (agg_g + agg_s)
    return _tc_combine(agg, x, W_in, b_in, W_out, b_out, W_root, b_root)


# speedup vs baseline: 2.2280x; 2.2280x over previous
import functools

import jax
import jax.numpy as jnp
from jax import lax
from jax.experimental import pallas as pl
from jax.experimental.pallas import tpu as pltpu
from jax.experimental.pallas import tpu_sc as plsc

N_NODES = 10000
D = 256
HALF = 128
ROWW = 144
NC = 2
NS = 16
B = 32
NBUF = 4
N_ACC = 10240
CHUNK = N_ACC // NS


def _sc_segment_sums(xcat, garr, tarr, zer, ept):
    n_batches = ept // B
    n_groups = n_batches // NBUF
    mesh = plsc.VectorSubcoreMesh(core_axis_name="c", subcore_axis_name="s")

    @functools.partial(
        pl.kernel,
        out_type=jax.ShapeDtypeStruct((2, NC, N_ACC, ROWW), jnp.float32),
        mesh=mesh,
        scratch_types=[
            pltpu.VMEM_SHARED((N_ACC, ROWW), jnp.float32),
            pltpu.VMEM((ept,), jnp.int32),
            pltpu.VMEM((ept,), jnp.int32),
        ] + [pltpu.VMEM((B, ROWW), jnp.float32) for _ in range(NBUF)]
          + [pltpu.SemaphoreType.DMA for _ in range(2 * NBUF)],
        compiler_params=pltpu.CompilerParams(use_tc_tiling_on_sc=False),
    )
    def k(xcat_hbm, garr_hbm, tarr_hbm, zer_hbm, out_hbm,
          acc_sh, gidx_f, tgt_f, *bufs_and_sems):
        rows = bufs_and_sems[:NBUF]
        sg = bufs_and_sems[NBUF:2 * NBUF]
        ss = bufs_and_sems[2 * NBUF:]
        c = lax.axis_index("c")
        s = lax.axis_index("s")
        row0 = pl.multiple_of(s * CHUNK, 8)
        pltpu.sync_copy(zer_hbm.at[pl.ds(0, CHUNK)],
                        acc_sh.at[pl.ds(row0, CHUNK)])
        plsc.subcore_barrier()
        for d in range(2):
            pltpu.sync_copy(garr_hbm.at[d, c, s], gidx_f)
            pltpu.sync_copy(tarr_hbm.at[d, s], tgt_f)

            for j in range(NBUF):
                pltpu.async_copy(
                    xcat_hbm.at[gidx_f.at[pl.ds(j * B, B)]], rows[j], sg[j])

            def group(i, carry):
                base = i * (NBUF * B)
                cs = []
                for j in range(NBUF):
                    pltpu.make_async_copy(
                        xcat_hbm.at[gidx_f.at[pl.ds(base + j * B, B)]],
                        rows[j], sg[j]).wait()
                    for kk in range(B // 16):
                        tv = tgt_f[pl.ds(base + j * B + kk * 16, 16)]
                        cs.append(pltpu.async_copy(
                            rows[j].at[pl.ds(kk * 16, 16)],
                            acc_sh.at[tv], ss[j], add=True))
                for j in range(NBUF):
                    for kk in range(B // 16):
                        cs[j * (B // 16) + kk].wait()

                    @pl.when(i < n_groups - 1)
                    def _():
                        pltpu.async_copy(
                            xcat_hbm.at[gidx_f.at[
                                pl.ds(base + (NBUF + j) * B, B)]],
                            rows[j], sg[j])
                return carry

            lax.fori_loop(0, n_groups, group, 0)
            plsc.subcore_barrier()
            pltpu.sync_copy(acc_sh.at[pl.ds(row0, CHUNK)],
                            out_hbm.at[d, c, pl.ds(row0, CHUNK)])
            if d == 0:
                pltpu.sync_copy(zer_hbm.at[pl.ds(0, CHUNK)],
                                acc_sh.at[pl.ds(row0, CHUNK)])
                plsc.subcore_barrier()

    return k(xcat, garr, tarr, zer)


def _tc_combine(agg, x, W_in, b_in, W_out, b_out, W_root, b_root):
    blk = 400
    grid = (N_NODES // blk,)

    def body(agg_ref, x_ref, wi_ref, wo_ref, wr_ref, b_ref, o_ref):
        dot = functools.partial(
            lax.dot_general,
            dimension_numbers=(((1,), (0,)), ((), ())),
            precision=lax.Precision.HIGHEST,
            preferred_element_type=jnp.float32,
        )
        acc = dot(x_ref[...], wr_ref[...])
        for d, w_ref in ((0, wi_ref), (1, wo_ref)):
            cnt = agg_ref[d, 0, :, HALF:HALF + 1]
            scale = 0.5 / jnp.maximum(cnt, 1.0)
            for c in range(2):
                h = agg_ref[d, c, :, 0:HALF] * scale
                acc = acc + dot(h, w_ref[c * HALF:(c + 1) * HALF, :])
        o_ref[...] = acc + b_ref[...]

    bias = (0.5 * b_in + 0.5 * b_out + b_root).reshape(1, D)
    return pl.pallas_call(
        body,
        grid=grid,
        in_specs=[
            pl.BlockSpec((2, NC, blk, ROWW), lambda i: (0, 0, i, 0)),
            pl.BlockSpec((blk, D), lambda i: (i, 0)),
            pl.BlockSpec((D, D), lambda i: (0, 0)),
            pl.BlockSpec((D, D), lambda i: (0, 0)),
            pl.BlockSpec((D, D), lambda i: (0, 0)),
            pl.BlockSpec((1, D), lambda i: (0, 0)),
        ],
        out_specs=pl.BlockSpec((blk, D), lambda i: (i, 0)),
        out_shape=jax.ShapeDtypeStruct((N_NODES, D), jnp.float32),
    )(agg, x, W_in, W_out, W_root, bias)


def kernel(x, edge_index, W_in, b_in, W_out, b_out, W_root, b_root):
    n_edges = edge_index.shape[1]
    src = edge_index[0].astype(jnp.int32)
    dst = edge_index[1].astype(jnp.int32)

    ones = jnp.ones((N_NODES, 1), jnp.float32)
    zpad = jnp.zeros((N_NODES, ROWW - HALF - 1), jnp.float32)
    xcat = jnp.concatenate([
        jnp.concatenate([x[:, :HALF], ones, zpad], axis=1),
        jnp.concatenate([x[:, HALF:], ones, zpad], axis=1),
        jnp.zeros((8, ROWW), jnp.float32),
    ], axis=0)

    ept = ((n_edges // NS) + NBUF * B - 1) // (NBUF * B) * (NBUF * B)
    e_pad = NS * ept
    pad = e_pad - n_edges
    pad_g = 2 * N_NODES + (jnp.arange(pad, dtype=jnp.int32) % 8)
    pad_t = jnp.arange(pad, dtype=jnp.int32) % N_NODES

    def tiled(idx, pad_vals):
        return jnp.concatenate([idx, pad_vals]).reshape(NS, ept)

    garr = jnp.stack([
        jnp.stack([tiled(src, pad_g), tiled(src + N_NODES, pad_g)]),
        jnp.stack([tiled(dst, pad_g), tiled(dst + N_NODES, pad_g)]),
    ])
    tarr = jnp.stack([tiled(dst, pad_t), tiled(src, pad_t)])

    zer = jnp.zeros((CHUNK, ROWW), jnp.float32)
    agg = _sc_segment_sums(xcat, garr, tarr, zer, ept)
    return _tc_combine(agg, x, W_in, b_in, W_out, b_out, W_root, b_root)


# speedup vs baseline: 2.7052x; 1.2142x over previous
import functools

import jax
import jax.numpy as jnp
from jax import lax
from jax.experimental import pallas as pl
from jax.experimental.pallas import tpu as pltpu
from jax.experimental.pallas import tpu_sc as plsc

N_NODES = 10000
D = 256
HALF = 128
CW = 16
NC = 2
NS = 16
B = 32
NBUF = 4
N_ACC = 10240
CHUNK = N_ACC // NS


def _sc_segment_sums(xfeat, garr, tarr, zf, zc, onesb, ept):
    n_batches = ept // B
    n_groups = n_batches // NBUF
    n_sub = B // 16
    mesh = plsc.VectorSubcoreMesh(core_axis_name="c", subcore_axis_name="s")

    @functools.partial(
        pl.kernel,
        out_type=(
            jax.ShapeDtypeStruct((2, NC, N_ACC, HALF), jnp.float32),
            jax.ShapeDtypeStruct((2, NC, N_ACC, CW), jnp.float32),
        ),
        mesh=mesh,
        scratch_types=[
            pltpu.VMEM_SHARED((N_ACC, HALF), jnp.float32),
            pltpu.VMEM_SHARED((N_ACC, CW), jnp.float32),
            pltpu.VMEM((ept,), jnp.int32),
            pltpu.VMEM((ept,), jnp.int32),
            pltpu.VMEM((16, CW), jnp.float32),
        ] + [pltpu.VMEM((B, HALF), jnp.float32) for _ in range(NBUF)]
          + [pltpu.SemaphoreType.DMA for _ in range(2 * NBUF)]
          + [pltpu.SemaphoreType.DMA],
        compiler_params=pltpu.CompilerParams(use_tc_tiling_on_sc=False),
    )
    def k(xfeat_hbm, garr_hbm, tarr_hbm, zf_hbm, zc_hbm, onesb_hbm,
          outf_hbm, outc_hbm,
          accf_sh, accc_sh, gidx_f, tgt_f, cntsrc, *bufs_and_sems):
        rows = bufs_and_sems[:NBUF]
        sg = bufs_and_sems[NBUF:2 * NBUF]
        ss = bufs_and_sems[2 * NBUF:3 * NBUF]
        scnt = bufs_and_sems[3 * NBUF]
        c = lax.axis_index("c")
        s = lax.axis_index("s")
        row0 = pl.multiple_of(s * CHUNK, 8)
        pltpu.sync_copy(onesb_hbm, cntsrc)
        pltpu.sync_copy(zf_hbm.at[pl.ds(0, CHUNK)],
                        accf_sh.at[pl.ds(row0, CHUNK)])
        pltpu.sync_copy(zc_hbm.at[pl.ds(0, CHUNK)],
                        accc_sh.at[pl.ds(row0, CHUNK)])
        plsc.subcore_barrier()
        for d in range(2):
            pltpu.sync_copy(garr_hbm.at[d, c, s], gidx_f)
            pltpu.sync_copy(tarr_hbm.at[d, s], tgt_f)

            for j in range(NBUF):
                pltpu.async_copy(
                    xfeat_hbm.at[gidx_f.at[pl.ds(j * B, B)]], rows[j], sg[j])

            def group(i, carry):
                base = i * (NBUF * B)
                cs = []
                for j in range(NBUF):
                    pltpu.make_async_copy(
                        xfeat_hbm.at[gidx_f.at[pl.ds(base + j * B, B)]],
                        rows[j], sg[j]).wait()
                    for kk in range(n_sub):
                        tv = tgt_f[pl.ds(base + j * B + kk * 16, 16)]
                        cs.append(pltpu.async_copy(
                            rows[j].at[pl.ds(kk * 16, 16)],
                            accf_sh.at[tv], ss[j], add=True))
                        pltpu.async_copy(cntsrc, accc_sh.at[tv], scnt,
                                         add=True)
                for j in range(NBUF):
                    for kk in range(n_sub):
                        cs[j * n_sub + kk].wait()

                    @pl.when(i < n_groups - 1)
                    def _():
                        pltpu.async_copy(
                            xfeat_hbm.at[gidx_f.at[
                                pl.ds(base + (NBUF + j) * B, B)]],
                            rows[j], sg[j])
                return carry

            lax.fori_loop(0, n_groups, group, 0)

            def drain(i, carry):
                pltpu.make_async_copy(onesb_hbm, cntsrc, scnt).wait()
                return carry

            lax.fori_loop(0, n_batches * n_sub, drain, 0)
            plsc.subcore_barrier()
            pltpu.sync_copy(accf_sh.at[pl.ds(row0, CHUNK)],
                            outf_hbm.at[d, c, pl.ds(row0, CHUNK)])
            pltpu.sync_copy(accc_sh.at[pl.ds(row0, CHUNK)],
                            outc_hbm.at[d, c, pl.ds(row0, CHUNK)])
            if d == 0:
                pltpu.sync_copy(zf_hbm.at[pl.ds(0, CHUNK)],
                                accf_sh.at[pl.ds(row0, CHUNK)])
                pltpu.sync_copy(zc_hbm.at[pl.ds(0, CHUNK)],
                                accc_sh.at[pl.ds(row0, CHUNK)])
                plsc.subcore_barrier()

    return k(xfeat, garr, tarr, zf, zc, onesb)


def _tc_combine(aggf, aggc, x, W_in, b_in, W_out, b_out, W_root, b_root):
    blk = 400
    grid = (N_NODES // blk,)

    def body(aggf_ref, aggc_ref, x_ref, wi_ref, wo_ref, wr_ref, b_ref,
             o_ref):
        dot = functools.partial(
            lax.dot_general,
            dimension_numbers=(((1,), (0,)), ((), ())),
            precision=lax.Precision.HIGHEST,
            preferred_element_type=jnp.float32,
        )
        acc = dot(x_ref[...], wr_ref[...])
        for d, w_ref in ((0, wi_ref), (1, wo_ref)):
            cnt = aggc_ref[d, 0, :, 0:1]
            scale = 0.5 / jnp.maximum(cnt, 1.0)
            for c in range(2):
                h = aggf_ref[d, c, :, :] * scale
                acc = acc + dot(h, w_ref[c * HALF:(c + 1) * HALF, :])
        o_ref[...] = acc + b_ref[...]

    bias = (0.5 * b_in + 0.5 * b_out + b_root).reshape(1, D)
    return pl.pallas_call(
        body,
        grid=grid,
        in_specs=[
            pl.BlockSpec((2, NC, blk, HALF), lambda i: (0, 0, i, 0)),
            pl.BlockSpec((2, NC, blk, CW), lambda i: (0, 0, i, 0)),
            pl.BlockSpec((blk, D), lambda i: (i, 0)),
            pl.BlockSpec((D, D), lambda i: (0, 0)),
            pl.BlockSpec((D, D), lambda i: (0, 0)),
            pl.BlockSpec((D, D), lambda i: (0, 0)),
            pl.BlockSpec((1, D), lambda i: (0, 0)),
        ],
        out_specs=pl.BlockSpec((blk, D), lambda i: (i, 0)),
        out_shape=jax.ShapeDtypeStruct((N_NODES, D), jnp.float32),
    )(aggf, aggc, x, W_in, W_out, W_root, bias)


def kernel(x, edge_index, W_in, b_in, W_out, b_out, W_root, b_root):
    n_edges = edge_index.shape[1]
    src = edge_index[0].astype(jnp.int32)
    dst = edge_index[1].astype(jnp.int32)

    xfeat = jnp.concatenate([x[:, :HALF], x[:, HALF:]], axis=0)

    ept = ((n_edges // NS) + NBUF * B - 1) // (NBUF * B) * (NBUF * B)
    e_pad = NS * ept
    pad = e_pad - n_edges
    pad_g = jnp.arange(pad, dtype=jnp.int32) % N_NODES
    pad_t = N_NODES + (jnp.arange(pad, dtype=jnp.int32) % (N_ACC - N_NODES))

    def tiled(idx, pad_vals):
        return jnp.concatenate([idx, pad_vals]).reshape(NS, ept)

    garr = jnp.stack([
        jnp.stack([tiled(src, pad_g), tiled(src + N_NODES, pad_g)]),
        jnp.stack([tiled(dst, pad_g), tiled(dst + N_NODES, pad_g)]),
    ])
    tarr = jnp.stack([tiled(dst, pad_t), tiled(src, pad_t)])

    zf = jnp.zeros((CHUNK, HALF), jnp.float32)
    zc = jnp.zeros((CHUNK, CW), jnp.float32)
    onesb = jnp.concatenate(
        [jnp.ones((16, 1), jnp.float32), jnp.zeros((16, CW - 1), jnp.float32)],
        axis=1)
    aggf, aggc = _sc_segment_sums(xfeat, garr, tarr, zf, zc, onesb, ept)
    return _tc_combine(aggf, aggc, x, W_in, b_in, W_out, b_out, W_root,
                       b_root)


# speedup vs baseline: 2.7420x; 1.0136x over previous
import functools

import jax
import jax.numpy as jnp
from jax import lax
from jax.experimental import pallas as pl
from jax.experimental.pallas import tpu as pltpu
from jax.experimental.pallas import tpu_sc as plsc

N_NODES = 10000
D = 256
HALF = 128
CW = 16
NC = 2
NS = 16
B = 32
NBUF = 4
N_ACC = 10240
CHUNK = N_ACC // NS


def _sc_segment_sums(xfeat, garr, tarr, zf, zc, onesb, ept):
    n_batches = ept // B
    n_groups = n_batches // NBUF
    n_sub = B // 16
    mesh = plsc.VectorSubcoreMesh(core_axis_name="c", subcore_axis_name="s")

    @functools.partial(
        pl.kernel,
        out_type=(
            jax.ShapeDtypeStruct((2, NC, N_ACC, HALF), jnp.float32),
            jax.ShapeDtypeStruct((2, NC, N_ACC, CW), jnp.float32),
        ),
        mesh=mesh,
        scratch_types=[
            pltpu.VMEM_SHARED((N_ACC, HALF), jnp.float32),
            pltpu.VMEM_SHARED((N_ACC, CW), jnp.float32),
            pltpu.VMEM((ept,), jnp.int32),
            pltpu.VMEM((ept,), jnp.int32),
            pltpu.VMEM((16, CW), jnp.float32),
        ] + [pltpu.VMEM((B, HALF), jnp.float32) for _ in range(NBUF)]
          + [pltpu.SemaphoreType.DMA for _ in range(2 * NBUF)]
          + [pltpu.SemaphoreType.DMA],
        compiler_params=pltpu.CompilerParams(use_tc_tiling_on_sc=False),
    )
    def k(xfeat_hbm, garr_hbm, tarr_hbm, zf_hbm, zc_hbm, onesb_hbm,
          outf_hbm, outc_hbm,
          accf_sh, accc_sh, gidx_f, tgt_f, cntsrc, *bufs_and_sems):
        rows = bufs_and_sems[:NBUF]
        sg = bufs_and_sems[NBUF:2 * NBUF]
        ss = bufs_and_sems[2 * NBUF:3 * NBUF]
        scnt = bufs_and_sems[3 * NBUF]
        c = lax.axis_index("c")
        s = lax.axis_index("s")
        row0 = pl.multiple_of(s * CHUNK, 8)
        pltpu.sync_copy(onesb_hbm, cntsrc)
        pltpu.sync_copy(zf_hbm.at[pl.ds(0, CHUNK)],
                        accf_sh.at[pl.ds(row0, CHUNK)])
        pltpu.sync_copy(zc_hbm.at[pl.ds(0, CHUNK)],
                        accc_sh.at[pl.ds(row0, CHUNK)])
        plsc.subcore_barrier()
        for d in range(2):
            pltpu.sync_copy(garr_hbm.at[d, c, s], gidx_f)
            pltpu.sync_copy(tarr_hbm.at[d, s], tgt_f)

            for j in range(NBUF):
                pltpu.async_copy(
                    xfeat_hbm.at[gidx_f.at[pl.ds(j * B, B)]], rows[j], sg[j])

            def group(i, carry):
                base = i * (NBUF * B)
                cs = []
                for j in range(NBUF):
                    pltpu.make_async_copy(
                        xfeat_hbm.at[gidx_f.at[pl.ds(base + j * B, B)]],
                        rows[j], sg[j]).wait()
                    for kk in range(n_sub):
                        tv = tgt_f[pl.ds(base + j * B + kk * 16, 16)]
                        cs.append(pltpu.async_copy(
                            rows[j].at[pl.ds(kk * 16, 16)],
                            accf_sh.at[tv], ss[j], add=True))
                        pltpu.async_copy(cntsrc, accc_sh.at[tv], scnt,
                                         add=True)
                for j in range(NBUF):
                    for kk in range(n_sub):
                        cs[j * n_sub + kk].wait()

                    @pl.when(i < n_groups - 1)
                    def _():
                        pltpu.async_copy(
                            xfeat_hbm.at[gidx_f.at[
                                pl.ds(base + (NBUF + j) * B, B)]],
                            rows[j], sg[j])
                return carry

            lax.fori_loop(0, n_groups, group, 0)

            def drain(i, carry):
                pltpu.make_async_copy(onesb_hbm, cntsrc, scnt).wait()
                return carry

            lax.fori_loop(0, n_batches * n_sub, drain, 0)
            plsc.subcore_barrier()
            pltpu.sync_copy(accf_sh.at[pl.ds(row0, CHUNK)],
                            outf_hbm.at[d, c, pl.ds(row0, CHUNK)])
            pltpu.sync_copy(accc_sh.at[pl.ds(row0, CHUNK)],
                            outc_hbm.at[d, c, pl.ds(row0, CHUNK)])
            if d == 0:
                pltpu.sync_copy(zf_hbm.at[pl.ds(0, CHUNK)],
                                accf_sh.at[pl.ds(row0, CHUNK)])
                pltpu.sync_copy(zc_hbm.at[pl.ds(0, CHUNK)],
                                accc_sh.at[pl.ds(row0, CHUNK)])
                plsc.subcore_barrier()

    return k(xfeat, garr, tarr, zf, zc, onesb)


def _tc_root(x, W_root, bias):
    blk = 400
    grid = (N_NODES // blk,)

    def body(x_ref, wr_ref, b_ref, o_ref):
        o_ref[...] = lax.dot_general(
            x_ref[...], wr_ref[...],
            dimension_numbers=(((1,), (0,)), ((), ())),
            precision=lax.Precision.HIGHEST,
            preferred_element_type=jnp.float32,
        ) + b_ref[...]

    return pl.pallas_call(
        body,
        grid=grid,
        in_specs=[
            pl.BlockSpec((blk, D), lambda i: (i, 0)),
            pl.BlockSpec((D, D), lambda i: (0, 0)),
            pl.BlockSpec((1, D), lambda i: (0, 0)),
        ],
        out_specs=pl.BlockSpec((blk, D), lambda i: (i, 0)),
        out_shape=jax.ShapeDtypeStruct((N_NODES, D), jnp.float32),
    )(x, W_root, bias)


def _tc_combine(aggf, aggc, root, W_in, W_out):
    blk = 400
    grid = (N_NODES // blk,)

    def body(aggf_ref, aggc_ref, root_ref, wi_ref, wo_ref, o_ref):
        dot = functools.partial(
            lax.dot_general,
            dimension_numbers=(((1,), (0,)), ((), ())),
            precision=lax.Precision.HIGHEST,
            preferred_element_type=jnp.float32,
        )
        acc = root_ref[...]
        for d, w_ref in ((0, wi_ref), (1, wo_ref)):
            cnt = aggc_ref[d, 0, :, 0:1]
            scale = 0.5 / jnp.maximum(cnt, 1.0)
            for c in range(2):
                h = aggf_ref[d, c, :, :] * scale
                acc = acc + dot(h, w_ref[c * HALF:(c + 1) * HALF, :])
        o_ref[...] = acc

    return pl.pallas_call(
        body,
        grid=grid,
        in_specs=[
            pl.BlockSpec((2, NC, blk, HALF), lambda i: (0, 0, i, 0)),
            pl.BlockSpec((2, NC, blk, CW), lambda i: (0, 0, i, 0)),
            pl.BlockSpec((blk, D), lambda i: (i, 0)),
            pl.BlockSpec((D, D), lambda i: (0, 0)),
            pl.BlockSpec((D, D), lambda i: (0, 0)),
        ],
        out_specs=pl.BlockSpec((blk, D), lambda i: (i, 0)),
        out_shape=jax.ShapeDtypeStruct((N_NODES, D), jnp.float32),
    )(aggf, aggc, root, W_in, W_out)


def kernel(x, edge_index, W_in, b_in, W_out, b_out, W_root, b_root):
    n_edges = edge_index.shape[1]
    src = edge_index[0].astype(jnp.int32)
    dst = edge_index[1].astype(jnp.int32)

    xfeat = jnp.concatenate([x[:, :HALF], x[:, HALF:]], axis=0)

    ept = ((n_edges // NS) + NBUF * B - 1) // (NBUF * B) * (NBUF * B)
    e_pad = NS * ept
    pad = e_pad - n_edges
    pad_g = jnp.arange(pad, dtype=jnp.int32) % N_NODES
    pad_t = N_NODES + (jnp.arange(pad, dtype=jnp.int32) % (N_ACC - N_NODES))

    def tiled(idx, pad_vals):
        return jnp.concatenate([idx, pad_vals]).reshape(NS, ept)

    garr = jnp.stack([
        jnp.stack([tiled(src, pad_g), tiled(src + N_NODES, pad_g)]),
        jnp.stack([tiled(dst, pad_g), tiled(dst + N_NODES, pad_g)]),
    ])
    tarr = jnp.stack([tiled(dst, pad_t), tiled(src, pad_t)])

    zf = jnp.zeros((CHUNK, HALF), jnp.float32)
    zc = jnp.zeros((CHUNK, CW), jnp.float32)
    onesb = jnp.concatenate(
        [jnp.ones((16, 1), jnp.float32), jnp.zeros((16, CW - 1), jnp.float32)],
        axis=1)
    bias = (0.5 * b_in + 0.5 * b_out + b_root).reshape(1, D)
    aggf, aggc = _sc_segment_sums(xfeat, garr, tarr, zf, zc, onesb, ept)
    root = _tc_root(x, W_root, bias)
    return _tc_combine(aggf, aggc, root, W_in, W_out)
